# baseline (speedup 1.0000x reference)
import functools

import jax
import jax.numpy as jnp
from jax import lax
from jax.experimental import pallas as pl
from jax.experimental.pallas import tpu as pltpu
from jax.experimental.pallas import tpu_sc as plsc

NC, NS, L = 2, 16, 16
NW = NC * NS
B = 16384
D = 64
BPW = B // NW
CHUNK = 128
NCH = BPW // CHUNK
GROUPS = CHUNK // L

_mesh = plsc.VectorSubcoreMesh(
    core_axis_name="c", subcore_axis_name="s", num_cores=NC, num_subcores=NS)


@functools.partial(
    pl.kernel,
    out_type=jax.ShapeDtypeStruct((B,), jnp.float32),
    mesh=_mesh,
    compiler_params=pltpu.CompilerParams(
        needs_layout_passes=False, use_tc_tiling_on_sc=False),
    scratch_types=[
        pltpu.VMEM((NCH, CHUNK), jnp.int32),
        pltpu.VMEM((NCH, CHUNK), jnp.int32),
        pltpu.VMEM((CHUNK, D), jnp.float32),
        pltpu.VMEM((CHUNK, D), jnp.float32),
        pltpu.VMEM((CHUNK,), jnp.float32),
        pltpu.VMEM((CHUNK,), jnp.float32),
        pltpu.VMEM((96,), jnp.float32),
        pltpu.VMEM((L * L,), jnp.float32),
        pltpu.VMEM((BPW,), jnp.float32),
        pltpu.SemaphoreType.DMA,
    ],
)
def _afmp_sc(emb_hbm, bias_hbm, ia_hbm, ib_hbm, w_hbm, out_hbm,
             ia_v, ib_v, ra_v, rb_v, ba_v, bb_v, w_v, m_v, o_v, sem):
    wid = lax.axis_index("s") * NC + lax.axis_index("c")
    base = wid * BPW
    pltpu.sync_copy(ia_hbm.at[wid], ia_v)
    pltpu.sync_copy(ib_hbm.at[wid], ib_v)
    pltpu.sync_copy(w_hbm, w_v)
    wv = [w_v[pl.ds(c * L, L)] for c in range(D // L)]
    w64v = w_v[pl.ds(D, L)]
    b0v = w_v[pl.ds(D + L, L)]
    iota = lax.iota(jnp.int32, L)

    for j in range(NCH):
        cps = [
            pltpu.async_copy(emb_hbm.at[ia_v.at[j]], ra_v, sem),
            pltpu.async_copy(emb_hbm.at[ib_v.at[j]], rb_v, sem),
            pltpu.async_copy(bias_hbm.at[ia_v.at[j]], ba_v, sem),
            pltpu.async_copy(bias_hbm.at[ib_v.at[j]], bb_v, sem),
        ]
        for cp in cps:
            cp.wait()

        def group(g, _):
            rbase = g * L
            for r in range(L):
                row = rbase + r
                acc = ra_v[row, pl.ds(0, L)] * rb_v[row, pl.ds(0, L)] * wv[0]
                for c in range(1, D // L):
                    acc = acc + (ra_v[row, pl.ds(c * L, L)]
                                 * rb_v[row, pl.ds(c * L, L)] * wv[c])
                m_v[pl.ds(r * L, L)] = acc
            res = plsc.load_gather(m_v, [iota * L])
            for kcol in range(1, L):
                res = res + plsc.load_gather(m_v, [iota * L + kcol])
            x = res + (ba_v[pl.ds(rbase, L)] + bb_v[pl.ds(rbase, L)]) * w64v + b0v
            o_v[pl.ds(j * CHUNK + rbase, L)] = 1.0 / (1.0 + jnp.exp(-x))
            return 0

        lax.fori_loop(0, GROUPS, group, 0)

    pltpu.sync_copy(o_v, out_hbm.at[pl.ds(base, BPW)])


def kernel(drug_a, drug_b, emb_table, bias_table, dense_W, dense_b):
    ia = drug_a.astype(jnp.int32).reshape(NW, NCH, CHUNK)
    ib = drug_b.astype(jnp.int32).reshape(NW, NCH, CHUNK)
    w = dense_W[:, 0]
    wpack = jnp.concatenate([
        w[:D],
        jnp.full((L,), w[D], jnp.float32),
        jnp.full((L,), dense_b[0], jnp.float32),
    ])
    out = _afmp_sc(emb_table, bias_table[:, 0], ia, ib, wpack)
    return out.reshape(B, 1)
